# baseline (speedup 1.0000x reference)
import functools

import jax
import jax.numpy as jnp
from jax import lax
from jax.experimental import pallas as pl
from jax.experimental.pallas import tpu as pltpu
from jax.experimental.pallas import tpu_sc as plsc

_NC = 2
_NS = 16


@functools.partial(
    pl.kernel,
    mesh=plsc.VectorSubcoreMesh(core_axis_name="c", subcore_axis_name="s"),
    out_type=jax.ShapeDtypeStruct((16,), jnp.float32),
    scratch_types=[
        pltpu.VMEM((16,), jnp.float32),
    ],
)
def _sc_probe(anchor_hbm, pos_hbm, pt_hbm, pb_hbm, out_hbm, half_v):
    wid = lax.axis_index("s") * _NC + lax.axis_index("c")
    half_v[...] = jnp.full((16,), 0.5, dtype=jnp.float32)

    @pl.when(wid == 0)
    def _():
        pltpu.sync_copy(half_v, out_hbm)


def kernel(anchor_ids, positive_ids, table):
    pt = table[:500000].reshape(250000, 128)
    pb = table[500000:].reshape(250000, 128)
    out = _sc_probe(anchor_ids.astype(jnp.int32),
                    positive_ids.astype(jnp.int32), pt, pb)
    return out[0]
